# Initial kernel scaffold; baseline (speedup 1.0000x reference)
import functools

import jax
import jax.numpy as jnp
from jax import lax
from jax.experimental import pallas as pl
from jax.experimental.pallas import tpu as pltpu
from jax.experimental.pallas import tpu_sc as plsc

NV = 10000
D = 128
H = 64
NNZ = 320000
TR = 10240
NT = 16
B = 128
GPT = 157
SPT = GPT * B
NNZP = SPT * NT
ROWS_PT = TR // NT
WB = ROWS_PT // B
DUMP = 10000

_mesh = plsc.VectorSubcoreMesh(core_axis_name="c", subcore_axis_name="s")


def _zero_rows(buf, nrows, ncols):
    z = jnp.zeros((16,), jnp.float32)

    def body(r, _):
        for l in range(ncols // 16):
            buf[r, pl.ds(l * 16, 16)] = z
        return 0

    lax.fori_loop(0, nrows, body, 0)


@functools.partial(
    pl.kernel,
    out_type=jax.ShapeDtypeStruct((2 * TR,), jnp.float32),
    mesh=_mesh,
    scratch_types=[
        pltpu.VMEM((B,), jnp.int32),
        pltpu.VMEM((B,), jnp.float32),
        pltpu.VMEM_SHARED((TR,), jnp.float32),
        pltpu.SemaphoreType.DMA,
    ],
)
def _deg_kernel(didx, deg_out, idx_v, ones_v, acc, sem):
    c = lax.axis_index("c")
    t = lax.axis_index("s")
    base_r = t * ROWS_PT

    def fill(val):
        v = jnp.full((16,), val, jnp.float32)

        def body(i, _):
            ones_v[pl.ds(i * 16, 16)] = v
            return 0

        lax.fori_loop(0, B // 16, body, 0)

    fill(0.0)
    for k in range(WB):
        pltpu.sync_copy(ones_v, acc.at[pl.ds(base_r + k * B, B)])
    fill(1.0)
    plsc.subcore_barrier()

    ib = c * NNZP + t * SPT

    def body(g, _):
        pltpu.sync_copy(didx.at[pl.ds(ib + g * B, B)], idx_v)
        pltpu.sync_copy(ones_v, acc.at[idx_v], add=True)
        return 0

    lax.fori_loop(0, GPT, body, 0)
    plsc.subcore_barrier()
    pltpu.sync_copy(acc.at[pl.ds(base_r, ROWS_PT)],
                    deg_out.at[pl.ds(c * TR + base_r, ROWS_PT)])


@functools.partial(
    pl.kernel,
    out_type=(jax.ShapeDtypeStruct((2 * TR, H), jnp.float32),
              jax.ShapeDtypeStruct((2 * TR, H), jnp.float32)),
    mesh=_mesh,
    scratch_types=[
        pltpu.VMEM((B,), jnp.int32),
        pltpu.VMEM((B,), jnp.int32),
        pltpu.VMEM((B, H), jnp.float32),
        pltpu.VMEM((B, H), jnp.float32),
        pltpu.VMEM((ROWS_PT,), jnp.float32),
        pltpu.VMEM_SHARED((TR, H), jnp.float32),
        pltpu.VMEM_SHARED((TR, H), jnp.float32),
        pltpu.SemaphoreType.DMA,
    ],
)
def _conv_kernel(xs_tab, nidx2, eidx2, de_inv, he_tab, agg_tab,
                 gidx_v, sidx_v, rows_v, wb_v, de_v, he_acc, agg_acc, sem):
    c = lax.axis_index("c")
    t = lax.axis_index("s")
    base_r = t * ROWS_PT
    hbase = c * TR + base_r

    _zero_rows(wb_v, B, H)
    for k in range(WB):
        pltpu.sync_copy(wb_v, he_acc.at[pl.ds(base_r + k * B, B)])
        pltpu.sync_copy(wb_v, agg_acc.at[pl.ds(base_r + k * B, B)])
    pltpu.sync_copy(de_inv.at[pl.ds(base_r, ROWS_PT)], de_v)
    plsc.subcore_barrier()

    ibg = c * NNZP + t * SPT
    ibs = t * SPT

    def sweep(table, gref, sref, acc):
        def body(g, _):
            pltpu.sync_copy(gref.at[pl.ds(ibg + g * B, B)], gidx_v)
            pltpu.sync_copy(sref.at[pl.ds(ibs + g * B, B)], sidx_v)
            pltpu.async_copy(table.at[gidx_v], rows_v, sem).wait()
            pltpu.sync_copy(rows_v, acc.at[sidx_v], add=True)
            return 0

        lax.fori_loop(0, GPT, body, 0)

    sweep(xs_tab, nidx2, eidx2, he_acc)
    plsc.subcore_barrier()

    for k in range(WB):
        pltpu.sync_copy(he_acc.at[pl.ds(base_r + k * B, B)], wb_v)

        def srow(r, _):
            s = de_v[k * B + r]
            for l in range(H // 16):
                wb_v[r, pl.ds(l * 16, 16)] = wb_v[r, pl.ds(l * 16, 16)] * s
            return 0

        lax.fori_loop(0, B, srow, 0)
        pltpu.sync_copy(wb_v, he_tab.at[pl.ds(hbase + k * B, B)])
    plsc.subcore_barrier()

    sweep(he_tab, eidx2, nidx2, agg_acc)
    plsc.subcore_barrier()
    pltpu.sync_copy(agg_acc.at[pl.ds(base_r, ROWS_PT)],
                    agg_tab.at[pl.ds(hbase, ROWS_PT)])


def _tc_scales(deg2):
    def body(dref, oref):
        d = dref[...]
        safe = jnp.where(d > 0, d, 1.0)
        row = lax.broadcasted_iota(jnp.int32, (2 * TR // 128, 128), 0)
        oref[...] = jnp.where(row < TR // 128, lax.rsqrt(safe), 1.0 / safe)

    return pl.pallas_call(
        body,
        out_shape=jax.ShapeDtypeStruct((2 * TR // 128, 128), jnp.float32),
    )(deg2)


def _tc_xs(x, dvs_col):
    def body(xref, dref, oref):
        oref[...] = xref[...] * dref[...]

    return pl.pallas_call(
        body,
        grid=(5,),
        in_specs=[pl.BlockSpec((2000, D), lambda g: (g, 0)),
                  pl.BlockSpec((2000, 1), lambda g: (g, 0))],
        out_specs=pl.BlockSpec((2000, D), lambda g: (g, 0)),
        out_shape=jax.ShapeDtypeStruct((NV, D), jnp.float32),
    )(x, dvs_col)


def _tc_layer(xp, agg, dvs_col, W, b2d):
    def body(xref, aref, dref, wref, bref, o1, o2):
        a = aref[...] * dref[...]
        y = jnp.dot(a, wref[...], preferred_element_type=jnp.float32)
        xn = jnp.maximum(xref[...] + y + bref[...], 0.0)
        o1[...] = xn
        o2[...] = xn * dref[...]

    return pl.pallas_call(
        body,
        grid=(5,),
        in_specs=[pl.BlockSpec((2000, D), lambda g: (g, 0)),
                  pl.BlockSpec((2000, D), lambda g: (g, 0)),
                  pl.BlockSpec((2000, 1), lambda g: (g, 0)),
                  pl.BlockSpec((D, D), lambda g: (0, 0)),
                  pl.BlockSpec((1, D), lambda g: (0, 0))],
        out_specs=[pl.BlockSpec((2000, D), lambda g: (g, 0))] * 2,
        out_shape=(jax.ShapeDtypeStruct((NV, D), jnp.float32),) * 2,
    )(xp, agg, dvs_col, W, b2d)


def _pack(xs):
    z = jnp.zeros((TR - NV, H), jnp.float32)
    return jnp.concatenate([xs[:, :H], z, xs[:, H:], z], axis=0)


def _unpack(aggf):
    return jnp.concatenate([aggf[:NV], aggf[TR:TR + NV]], axis=1)


def kernel(node_features, incidence, W1, b1, W2, b2):
    x = node_features
    nidx = incidence[0]
    eidx = incidence[1]
    pad = jnp.full((NNZP - NNZ,), DUMP, jnp.int32)
    nidx_p = jnp.concatenate([nidx, pad])
    eidx_p = jnp.concatenate([eidx, pad])
    nidx2 = jnp.concatenate([nidx_p, nidx_p + TR])
    eidx2 = jnp.concatenate([eidx_p, eidx_p + TR])
    didx = jnp.concatenate([nidx_p, eidx_p])

    deg = _deg_kernel(didx)
    scales = _tc_scales(deg.reshape(2 * TR // 128, 128))
    sflat = scales.reshape(-1)
    dvs_col = sflat[:NV, None]
    de_flat = sflat[TR:]

    xs_tab = _pack(_tc_xs(x, dvs_col))
    _, agg1f = _conv_kernel(xs_tab, nidx2, eidx2, de_flat)
    x1, xs1 = _tc_layer(x, _unpack(agg1f), dvs_col, W1, b1.reshape(1, D))
    _, agg2f = _conv_kernel(_pack(xs1), nidx2, eidx2, de_flat)
    x2, _ = _tc_layer(x1, _unpack(agg2f), dvs_col, W2, b2.reshape(1, D))
    return x2


# speedup vs baseline: 3.9581x; 3.9581x over previous
import functools

import jax
import jax.numpy as jnp
from jax import lax
from jax.experimental import pallas as pl
from jax.experimental.pallas import tpu as pltpu
from jax.experimental.pallas import tpu_sc as plsc

NV = 10000
D = 128
NNZ = 320000
TR = 10240
NT = 16
B = 128
GC = 79
SPC = GC * B
P = 32 * SPC
GD = 158
SPD = GD * B
ROWS_PT = TR // NT
WB = ROWS_PT // B
DUMP = 10000
BLK = 2048

_mesh = plsc.VectorSubcoreMesh(core_axis_name="c", subcore_axis_name="s")


def _zero_rows(buf, nrows, ncols):
    z = jnp.zeros((16,), jnp.float32)

    def body(r, _):
        for l in range(ncols // 16):
            buf[r, pl.ds(l * 16, 16)] = z
        return 0

    lax.fori_loop(0, nrows, body, 0)


@functools.partial(
    pl.kernel,
    out_type=jax.ShapeDtypeStruct((2 * TR,), jnp.float32),
    mesh=_mesh,
    scratch_types=[
        pltpu.VMEM((B,), jnp.int32),
        pltpu.VMEM((B,), jnp.float32),
        pltpu.VMEM_SHARED((TR,), jnp.float32),
        pltpu.SemaphoreType.DMA,
    ],
)
def _deg_kernel(didx, deg_out, idx_v, ones_v, acc, sem):
    c = lax.axis_index("c")
    t = lax.axis_index("s")
    base_r = t * ROWS_PT

    def fill(val):
        v = jnp.full((16,), val, jnp.float32)

        def body(i, _):
            ones_v[pl.ds(i * 16, 16)] = v
            return 0

        lax.fori_loop(0, B // 16, body, 0)

    fill(0.0)
    for k in range(WB):
        pltpu.sync_copy(ones_v, acc.at[pl.ds(base_r + k * B, B)])
    fill(1.0)
    plsc.subcore_barrier()

    ib = c * P + t * SPD

    def body(g, _):
        pltpu.sync_copy(didx.at[pl.ds(ib + g * B, B)], idx_v)
        pltpu.sync_copy(ones_v, acc.at[idx_v], add=True)
        return 0

    lax.fori_loop(0, GD, body, 0)
    plsc.subcore_barrier()
    pltpu.sync_copy(acc.at[pl.ds(base_r, ROWS_PT)],
                    deg_out.at[pl.ds(c * TR + base_r, ROWS_PT)])


@functools.partial(
    pl.kernel,
    out_type=jax.ShapeDtypeStruct((2 * TR, D), jnp.float32),
    mesh=_mesh,
    scratch_types=[
        pltpu.VMEM((B,), jnp.int32),
        pltpu.VMEM((B,), jnp.int32),
        pltpu.VMEM((B, D), jnp.float32),
        pltpu.VMEM((B, D), jnp.float32),
        pltpu.VMEM_SHARED((TR, D), jnp.float32),
        pltpu.SemaphoreType.DMA,
    ],
)
def _sweep_kernel(table, gidx, sidx, part, gidx_v, sidx_v, rows_v, zb_v,
                  acc, sem):
    c = lax.axis_index("c")
    t = lax.axis_index("s")
    base_r = t * ROWS_PT

    _zero_rows(zb_v, B, D)
    for k in range(WB):
        pltpu.sync_copy(zb_v, acc.at[pl.ds(base_r + k * B, B)])
    plsc.subcore_barrier()

    ib = (c * NT + t) * SPC

    def body(g, _):
        pltpu.sync_copy(gidx.at[pl.ds(ib + g * B, B)], gidx_v)
        pltpu.sync_copy(sidx.at[pl.ds(ib + g * B, B)], sidx_v)
        pltpu.async_copy(table.at[gidx_v], rows_v, sem).wait()
        pltpu.sync_copy(rows_v, acc.at[sidx_v], add=True)
        return 0

    lax.fori_loop(0, GC, body, 0)
    plsc.subcore_barrier()
    pltpu.sync_copy(acc.at[pl.ds(base_r, ROWS_PT)],
                    part.at[pl.ds(c * TR + base_r, ROWS_PT)])


def _tc_scales(deg2):
    def body(dref, oref):
        d = dref[...]
        safe = jnp.where(d > 0, d, 1.0)
        row = lax.broadcasted_iota(jnp.int32, (2 * TR // 128, 128), 0)
        oref[...] = jnp.where(row < TR // 128, lax.rsqrt(safe), 1.0 / safe)

    return pl.pallas_call(
        body,
        out_shape=jax.ShapeDtypeStruct((2 * TR // 128, 128), jnp.float32),
    )(deg2)


def _tc_xs(x, dvs_col):
    def body(xref, dref, oref):
        oref[...] = xref[...] * dref[...]

    return pl.pallas_call(
        body,
        grid=(TR // BLK,),
        in_specs=[pl.BlockSpec((BLK, D), lambda g: (g, 0)),
                  pl.BlockSpec((BLK, 1), lambda g: (g, 0))],
        out_specs=pl.BlockSpec((BLK, D), lambda g: (g, 0)),
        out_shape=jax.ShapeDtypeStruct((TR, D), jnp.float32),
    )(x, dvs_col)


def _tc_hemerge(part, de_col):
    def body(aref, bref, dref, oref):
        oref[...] = (aref[...] + bref[...]) * dref[...]

    return pl.pallas_call(
        body,
        grid=(TR // BLK,),
        in_specs=[pl.BlockSpec((BLK, D), lambda g: (g, 0)),
                  pl.BlockSpec((BLK, D), lambda g: (g + TR // BLK, 0)),
                  pl.BlockSpec((BLK, 1), lambda g: (g, 0))],
        out_specs=pl.BlockSpec((BLK, D), lambda g: (g, 0)),
        out_shape=jax.ShapeDtypeStruct((TR, D), jnp.float32),
    )(part, part, de_col)


def _tc_layer(xp, agg_part, dvs_col, W, b2d):
    def body(xref, aref, bref, dref, wref, biasref, o1, o2):
        a = (aref[...] + bref[...]) * dref[...]
        y = jnp.dot(a, wref[...], preferred_element_type=jnp.float32)
        xn = jnp.maximum(xref[...] + y + biasref[...], 0.0)
        o1[...] = xn
        o2[...] = xn * dref[...]

    return pl.pallas_call(
        body,
        grid=(TR // BLK,),
        in_specs=[pl.BlockSpec((BLK, D), lambda g: (g, 0)),
                  pl.BlockSpec((BLK, D), lambda g: (g, 0)),
                  pl.BlockSpec((BLK, D), lambda g: (g + TR // BLK, 0)),
                  pl.BlockSpec((BLK, 1), lambda g: (g, 0)),
                  pl.BlockSpec((D, D), lambda g: (0, 0)),
                  pl.BlockSpec((1, D), lambda g: (0, 0))],
        out_specs=[pl.BlockSpec((BLK, D), lambda g: (g, 0))] * 2,
        out_shape=(jax.ShapeDtypeStruct((TR, D), jnp.float32),) * 2,
    )(xp, agg_part, agg_part, dvs_col, W, b2d)


def kernel(node_features, incidence, W1, b1, W2, b2):
    nidx = incidence[0]
    eidx = incidence[1]
    pad = jnp.full((P - NNZ,), DUMP, jnp.int32)
    nidx_p = jnp.concatenate([nidx, pad])
    eidx_p = jnp.concatenate([eidx, pad])
    didx = jnp.concatenate([nidx_p, eidx_p])
    x_pad = jnp.concatenate(
        [node_features, jnp.zeros((TR - NV, D), jnp.float32)], axis=0)

    deg = _deg_kernel(didx)
    scales = _tc_scales(deg.reshape(2 * TR // 128, 128))
    sflat = scales.reshape(-1)
    dvs_col = sflat[:TR, None]
    de_col = sflat[TR:, None]

    xs = _tc_xs(x_pad, dvs_col)
    xp = x_pad
    for (W, b) in ((W1, b1), (W2, b2)):
        he_part = _sweep_kernel(xs, nidx_p, eidx_p)
        he = _tc_hemerge(he_part, de_col)
        agg_part = _sweep_kernel(he, eidx_p, nidx_p)
        xp, xs = _tc_layer(xp, agg_part, dvs_col, W, b.reshape(1, D))
    return xp[:NV]
